# baseline (speedup 1.0000x reference)
import functools

import jax
import jax.numpy as jnp
from jax import lax
from jax.experimental import pallas as pl
from jax.experimental.pallas import tpu as pltpu
from jax.experimental.pallas import tpu_sc as plsc

N = 10000
E = 320000
D = 128
H = 128

NC = 2
NS = 16
NW = NC * NS
L = 16

ET = E // NW
K = 128
NF = ET // K
TAIL = ET - NF * K
RPT = N // NS
BR = 1000
BF = jnp.bfloat16


def _sc_mesh():
    return plsc.VectorSubcoreMesh(core_axis_name="c", subcore_axis_name="s",
                                  num_cores=NC, num_subcores=NS)


@functools.partial(
    pl.kernel,
    mesh=_sc_mesh(),
    out_type=jax.ShapeDtypeStruct((NC, N, L), jnp.float32),
    scratch_types=[
        pltpu.VMEM_SHARED((N, L), jnp.float32),
        pltpu.VMEM((ET,), jnp.int32),
        pltpu.VMEM((K, L), jnp.float32),
        pltpu.VMEM((RPT, L), jnp.float32),
        pltpu.SemaphoreType.DMA,
    ],
    compiler_params=pltpu.CompilerParams(use_tc_tiling_on_sc=False),
)
def _deg_kernel(edges_hbm, out_hbm, acc_sh, idx_v, ones_v, z_v, sem):
    c = lax.axis_index("c")
    s = lax.axis_index("s")
    tile = c * NS + s

    def fill_z(i, _):
        z_v[i] = jnp.zeros((L,), jnp.float32)
        return 0

    lax.fori_loop(0, RPT, fill_z, 0)

    def fill_ones(i, _):
        ones_v[i] = jnp.full((L,), 1.0, jnp.float32)
        return 0

    lax.fori_loop(0, K, fill_ones, 0)

    pltpu.sync_copy(z_v, acc_sh.at[pl.ds(s * RPT, RPT)])
    plsc.subcore_barrier()

    pltpu.sync_copy(edges_hbm.at[1, pl.ds(tile * ET, ET)], idx_v)

    def body(j, _):
        pltpu.sync_copy(ones_v, acc_sh.at[idx_v.at[pl.ds(j * K, K)]], add=True)
        return 0

    lax.fori_loop(0, ET // K, body, 0)
    pltpu.sync_copy(ones_v.at[pl.ds(0, ET - (ET // K) * K)],
                    acc_sh.at[idx_v.at[pl.ds((ET // K) * K, ET - (ET // K) * K)]],
                    add=True)
    plsc.subcore_barrier()

    pltpu.sync_copy(acc_sh.at[pl.ds(s * RPT, RPT)],
                    out_hbm.at[c, pl.ds(s * RPT, RPT)])


@functools.partial(
    pl.kernel,
    mesh=_sc_mesh(),
    out_type=jax.ShapeDtypeStruct((NC, N, H), BF),
    scratch_types=[
        pltpu.VMEM_SHARED((N, H), BF),
        pltpu.VMEM((ET,), jnp.int32),
        pltpu.VMEM((ET,), jnp.int32),
        pltpu.VMEM((K, H), BF),
        pltpu.VMEM((K, H), BF),
        pltpu.SemaphoreType.DMA,
        pltpu.SemaphoreType.DMA,
    ],
    compiler_params=pltpu.CompilerParams(use_tc_tiling_on_sc=False),
)
def _agg_kernel(edges_hbm, y_hbm, out_hbm,
                acc_sh, src_v, dst_v, rows0, rows1, semA, semB):
    c = lax.axis_index("c")
    s = lax.axis_index("s")
    tile = c * NS + s

    def fill_row(i, _):
        def fill_lane(k, _):
            rows0[i, pl.ds(k * 2 * L, 2 * L)] = jnp.zeros((2 * L,), BF)
            return 0

        lax.fori_loop(0, H // (2 * L), fill_lane, 0)
        return 0

    lax.fori_loop(0, K, fill_row, 0)

    def zcopy(m, _):
        pltpu.sync_copy(rows0, acc_sh.at[pl.ds(s * RPT + m * K, K)])
        return 0

    lax.fori_loop(0, RPT // K, zcopy, 0)
    pltpu.sync_copy(rows0.at[pl.ds(0, RPT - (RPT // K) * K)],
                    acc_sh.at[pl.ds(s * RPT + (RPT // K) * K,
                                    RPT - (RPT // K) * K)])
    plsc.subcore_barrier()

    pltpu.sync_copy(edges_hbm.at[0, pl.ds(tile * ET, ET)], src_v)
    pltpu.sync_copy(edges_hbm.at[1, pl.ds(tile * ET, ET)], dst_v)

    pltpu.async_copy(y_hbm.at[src_v.at[pl.ds(0, K)]], rows0, semA)

    def body(i, _):
        a = 2 * i
        pltpu.make_async_copy(
            y_hbm.at[src_v.at[pl.ds(a * K, K)]], rows0, semA).wait()
        pltpu.async_copy(
            y_hbm.at[src_v.at[pl.ds((a + 1) * K, K)]], rows1, semB)
        pltpu.sync_copy(rows0, acc_sh.at[dst_v.at[pl.ds(a * K, K)]],
                        add=True)
        pltpu.make_async_copy(
            y_hbm.at[src_v.at[pl.ds((a + 1) * K, K)]], rows1, semB).wait()

        @pl.when(a + 2 < NF)
        def _():
            pltpu.async_copy(
                y_hbm.at[src_v.at[pl.ds((a + 2) * K, K)]], rows0, semA)

        pltpu.sync_copy(rows1, acc_sh.at[dst_v.at[pl.ds((a + 1) * K, K)]],
                        add=True)
        return 0

    lax.fori_loop(0, NF // 2, body, 0)
    pltpu.async_copy(y_hbm.at[src_v.at[pl.ds(NF * K, TAIL)]],
                     rows0.at[pl.ds(0, TAIL)], semA)
    pltpu.make_async_copy(y_hbm.at[src_v.at[pl.ds(NF * K, TAIL)]],
                          rows0.at[pl.ds(0, TAIL)], semA).wait()
    pltpu.sync_copy(rows0.at[pl.ds(0, TAIL)],
                    acc_sh.at[dst_v.at[pl.ds(NF * K, TAIL)]], add=True)

    plsc.subcore_barrier()
    pltpu.sync_copy(acc_sh.at[pl.ds(s * RPT, RPT)],
                    out_hbm.at[c, pl.ds(s * RPT, RPT)])


def _mid_body(x_ref, w1_ref, degp_ref, y_ref):
    xw = jnp.dot(x_ref[...], w1_ref[...], preferred_element_type=jnp.float32)
    deg = degp_ref[0, :, 0:1] + degp_ref[1, :, 0:1] + 1.0
    dis = lax.rsqrt(deg)
    y_ref[...] = xw * dis


def _head_body(degp_ref, acc_ref, y_ref, b1_ref, w2_ref, b2_ref, out_ref):
    deg = degp_ref[0, :, 0:1] + degp_ref[1, :, 0:1] + 1.0
    dis = lax.rsqrt(deg)
    acc = acc_ref[...] + y_ref[...]
    h = jnp.maximum(dis * acc + b1_ref[...], 0.0)
    out_ref[...] = jnp.sum(h * w2_ref[...], axis=1, keepdims=True) + b2_ref[...]


def kernel(x, edge_index, W1, b1, W2, b2):
    deg_parts = _deg_kernel(edge_index)

    y = pl.pallas_call(
        _mid_body,
        grid=(N // BR,),
        in_specs=[
            pl.BlockSpec((BR, D), lambda i: (i, 0)),
            pl.BlockSpec((D, H), lambda i: (0, 0)),
            pl.BlockSpec((NC, BR, L), lambda i: (0, i, 0)),
        ],
        out_specs=pl.BlockSpec((BR, H), lambda i: (i, 0)),
        out_shape=jax.ShapeDtypeStruct((N, H), jnp.float32),
    )(x, W1, deg_parts)

    acc_parts = _agg_kernel(edge_index, y.astype(BF))
    acc = (acc_parts[0].astype(jnp.float32)
           + acc_parts[1].astype(jnp.float32))

    out2d = pl.pallas_call(
        _head_body,
        grid=(N // BR,),
        in_specs=[
            pl.BlockSpec((NC, BR, L), lambda i: (0, i, 0)),
            pl.BlockSpec((BR, H), lambda i: (i, 0)),
            pl.BlockSpec((BR, H), lambda i: (i, 0)),
            pl.BlockSpec((1, H), lambda i: (0, 0)),
            pl.BlockSpec((1, H), lambda i: (0, 0)),
            pl.BlockSpec((1, 1), lambda i: (0, 0)),
        ],
        out_specs=pl.BlockSpec((BR, 1), lambda i: (i, 0)),
        out_shape=jax.ShapeDtypeStruct((N, 1), jnp.float32),
    )(deg_parts, acc, y, b1.reshape(1, H), W2.reshape(1, H),
      b2.reshape(1, 1))

    return out2d.reshape(N)


# speedup vs baseline: 1.2292x; 1.2292x over previous
import functools

import jax
import jax.numpy as jnp
from jax import lax
from jax.experimental import pallas as pl
from jax.experimental.pallas import tpu as pltpu
from jax.experimental.pallas import tpu_sc as plsc

N = 10000
E = 320000
D = 128
H = 128

NC = 2
NS = 16
NW = NC * NS
L = 16

ET = E // NW
K = 128
NF = ET // K
TAIL = ET - NF * K
RPT = N // NS
BR = 1000
BF = jnp.bfloat16


def _sc_mesh():
    return plsc.VectorSubcoreMesh(core_axis_name="c", subcore_axis_name="s",
                                  num_cores=NC, num_subcores=NS)


@functools.partial(
    pl.kernel,
    mesh=_sc_mesh(),
    out_type=jax.ShapeDtypeStruct((NC, N, L), jnp.float32),
    scratch_types=[
        pltpu.VMEM_SHARED((N, L), jnp.float32),
        pltpu.VMEM((ET,), jnp.int32),
        pltpu.VMEM((K, L), jnp.float32),
        pltpu.VMEM((RPT, L), jnp.float32),
        pltpu.SemaphoreType.DMA,
    ],
    compiler_params=pltpu.CompilerParams(use_tc_tiling_on_sc=False),
)
def _deg_kernel(edges_hbm, out_hbm, acc_sh, idx_v, ones_v, z_v, sem):
    c = lax.axis_index("c")
    s = lax.axis_index("s")
    tile = c * NS + s

    def fill_z(i, _):
        z_v[i] = jnp.zeros((L,), jnp.float32)
        return 0

    lax.fori_loop(0, RPT, fill_z, 0)

    def fill_ones(i, _):
        ones_v[i] = jnp.full((L,), 1.0, jnp.float32)
        return 0

    lax.fori_loop(0, K, fill_ones, 0)

    pltpu.sync_copy(z_v, acc_sh.at[pl.ds(s * RPT, RPT)])
    plsc.subcore_barrier()

    pltpu.sync_copy(edges_hbm.at[1, pl.ds(tile * ET, ET)], idx_v)

    def body(j, _):
        pltpu.sync_copy(ones_v, acc_sh.at[idx_v.at[pl.ds(j * K, K)]], add=True)
        return 0

    lax.fori_loop(0, ET // K, body, 0)
    pltpu.sync_copy(ones_v.at[pl.ds(0, ET - (ET // K) * K)],
                    acc_sh.at[idx_v.at[pl.ds((ET // K) * K, ET - (ET // K) * K)]],
                    add=True)
    plsc.subcore_barrier()

    pltpu.sync_copy(acc_sh.at[pl.ds(s * RPT, RPT)],
                    out_hbm.at[c, pl.ds(s * RPT, RPT)])


@functools.partial(
    pl.kernel,
    mesh=_sc_mesh(),
    out_type=jax.ShapeDtypeStruct((NC, N, H), BF),
    scratch_types=[
        pltpu.VMEM_SHARED((N, H), BF),
        pltpu.VMEM((ET,), jnp.int32),
        pltpu.VMEM((ET,), jnp.int32),
        pltpu.VMEM((K, H), BF),
        pltpu.VMEM((K, H), BF),
        pltpu.VMEM((K, H), BF),
        pltpu.VMEM((K, H), BF),
        pltpu.SemaphoreType.DMA,
        pltpu.SemaphoreType.DMA,
        pltpu.SemaphoreType.DMA,
        pltpu.SemaphoreType.DMA,
    ],
    compiler_params=pltpu.CompilerParams(use_tc_tiling_on_sc=False),
)
def _agg_kernel(edges_hbm, y_hbm, out_hbm,
                acc_sh, src_v, dst_v, rows0, rows1, rows2, rows3,
                semA, semB, semC, semD):
    c = lax.axis_index("c")
    s = lax.axis_index("s")
    tile = c * NS + s

    def fill_row(i, _):
        def fill_lane(k, _):
            rows0[i, pl.ds(k * 2 * L, 2 * L)] = jnp.zeros((2 * L,), BF)
            return 0

        lax.fori_loop(0, H // (2 * L), fill_lane, 0)
        return 0

    lax.fori_loop(0, K, fill_row, 0)

    def zcopy(m, _):
        pltpu.sync_copy(rows0, acc_sh.at[pl.ds(s * RPT + m * K, K)])
        return 0

    lax.fori_loop(0, RPT // K, zcopy, 0)
    pltpu.sync_copy(rows0.at[pl.ds(0, RPT - (RPT // K) * K)],
                    acc_sh.at[pl.ds(s * RPT + (RPT // K) * K,
                                    RPT - (RPT // K) * K)])
    plsc.subcore_barrier()

    pltpu.sync_copy(edges_hbm.at[0, pl.ds(tile * ET, ET)], src_v)
    pltpu.sync_copy(edges_hbm.at[1, pl.ds(tile * ET, ET)], dst_v)

    bufs = (rows0, rows1, rows2, rows3)
    sems = (semA, semB, semC, semD)
    for b in range(3):
        pltpu.async_copy(y_hbm.at[src_v.at[pl.ds(b * K, K)]], bufs[b],
                         sems[b])

    def body(i, _):
        for b in range(4):
            a = 4 * i + b
            pltpu.make_async_copy(
                y_hbm.at[src_v.at[pl.ds(a * K, K)]], bufs[b], sems[b]).wait()

            @pl.when(a + 3 < NF)
            def _():
                pltpu.async_copy(
                    y_hbm.at[src_v.at[pl.ds((a + 3) * K, K)]],
                    bufs[(b + 3) % 4], sems[(b + 3) % 4])

            pltpu.sync_copy(bufs[b], acc_sh.at[dst_v.at[pl.ds(a * K, K)]],
                            add=True)
        return 0

    lax.fori_loop(0, NF // 4, body, 0)
    for a in range(4 * (NF // 4), NF):
        b = a % 4
        pltpu.make_async_copy(
            y_hbm.at[src_v.at[pl.ds(a * K, K)]], bufs[b], sems[b]).wait()
        pltpu.sync_copy(bufs[b], acc_sh.at[dst_v.at[pl.ds(a * K, K)]],
                        add=True)
    pltpu.async_copy(y_hbm.at[src_v.at[pl.ds(NF * K, TAIL)]],
                     rows0.at[pl.ds(0, TAIL)], semA)
    pltpu.make_async_copy(y_hbm.at[src_v.at[pl.ds(NF * K, TAIL)]],
                          rows0.at[pl.ds(0, TAIL)], semA).wait()
    pltpu.sync_copy(rows0.at[pl.ds(0, TAIL)],
                    acc_sh.at[dst_v.at[pl.ds(NF * K, TAIL)]], add=True)

    plsc.subcore_barrier()
    pltpu.sync_copy(acc_sh.at[pl.ds(s * RPT, RPT)],
                    out_hbm.at[c, pl.ds(s * RPT, RPT)])


def _mid_body(x_ref, w1_ref, degp_ref, y_ref):
    xw = jnp.dot(x_ref[...], w1_ref[...], preferred_element_type=jnp.float32)
    deg = degp_ref[0, :, 0:1] + degp_ref[1, :, 0:1] + 1.0
    dis = lax.rsqrt(deg)
    y_ref[...] = xw * dis


def _head_body(degp_ref, acc_ref, y_ref, b1_ref, w2_ref, b2_ref, out_ref):
    deg = degp_ref[0, :, 0:1] + degp_ref[1, :, 0:1] + 1.0
    dis = lax.rsqrt(deg)
    acc = acc_ref[...] + y_ref[...]
    h = jnp.maximum(dis * acc + b1_ref[...], 0.0)
    out_ref[...] = jnp.sum(h * w2_ref[...], axis=1, keepdims=True) + b2_ref[...]


def kernel(x, edge_index, W1, b1, W2, b2):
    deg_parts = _deg_kernel(edge_index)

    y = pl.pallas_call(
        _mid_body,
        grid=(N // BR,),
        in_specs=[
            pl.BlockSpec((BR, D), lambda i: (i, 0)),
            pl.BlockSpec((D, H), lambda i: (0, 0)),
            pl.BlockSpec((NC, BR, L), lambda i: (0, i, 0)),
        ],
        out_specs=pl.BlockSpec((BR, H), lambda i: (i, 0)),
        out_shape=jax.ShapeDtypeStruct((N, H), jnp.float32),
    )(x, W1, deg_parts)

    acc_parts = _agg_kernel(edge_index, y.astype(BF))
    acc = (acc_parts[0].astype(jnp.float32)
           + acc_parts[1].astype(jnp.float32))

    out2d = pl.pallas_call(
        _head_body,
        grid=(N // BR,),
        in_specs=[
            pl.BlockSpec((NC, BR, L), lambda i: (0, i, 0)),
            pl.BlockSpec((BR, H), lambda i: (i, 0)),
            pl.BlockSpec((BR, H), lambda i: (i, 0)),
            pl.BlockSpec((1, H), lambda i: (0, 0)),
            pl.BlockSpec((1, H), lambda i: (0, 0)),
            pl.BlockSpec((1, 1), lambda i: (0, 0)),
        ],
        out_specs=pl.BlockSpec((BR, 1), lambda i: (i, 0)),
        out_shape=jax.ShapeDtypeStruct((N, 1), jnp.float32),
    )(deg_parts, acc, y, b1.reshape(1, H), W2.reshape(1, H),
      b2.reshape(1, 1))

    return out2d.reshape(N)


# speedup vs baseline: 1.2646x; 1.0288x over previous
import functools

import jax
import jax.numpy as jnp
from jax import lax
from jax.experimental import pallas as pl
from jax.experimental.pallas import tpu as pltpu
from jax.experimental.pallas import tpu_sc as plsc

N = 10000
E = 320000
D = 128
H = 128

NC = 2
NS = 16
NW = NC * NS
L = 16

ET = E // NW
K = 128
NF = ET // K
TAIL = ET - NF * K
RPT = N // NS
BR = 2000
BF = jnp.bfloat16


def _sc_mesh():
    return plsc.VectorSubcoreMesh(core_axis_name="c", subcore_axis_name="s",
                                  num_cores=NC, num_subcores=NS)


@functools.partial(
    pl.kernel,
    mesh=_sc_mesh(),
    out_type=jax.ShapeDtypeStruct((NC, N, L), jnp.float32),
    scratch_types=[
        pltpu.VMEM_SHARED((N, L), jnp.float32),
        pltpu.VMEM((ET,), jnp.int32),
        pltpu.VMEM((K, L), jnp.float32),
        pltpu.VMEM((RPT, L), jnp.float32),
        pltpu.SemaphoreType.DMA,
    ],
    compiler_params=pltpu.CompilerParams(use_tc_tiling_on_sc=False),
)
def _deg_kernel(edges_hbm, out_hbm, acc_sh, idx_v, ones_v, z_v, sem):
    c = lax.axis_index("c")
    s = lax.axis_index("s")
    tile = c * NS + s

    def fill_z(i, _):
        z_v[i] = jnp.zeros((L,), jnp.float32)
        return 0

    lax.fori_loop(0, RPT, fill_z, 0)

    def fill_ones(i, _):
        ones_v[i] = jnp.full((L,), 1.0, jnp.float32)
        return 0

    lax.fori_loop(0, K, fill_ones, 0)

    pltpu.sync_copy(z_v, acc_sh.at[pl.ds(s * RPT, RPT)])
    plsc.subcore_barrier()

    pltpu.sync_copy(edges_hbm.at[1, pl.ds(tile * ET, ET)], idx_v)

    def body(j, _):
        pltpu.sync_copy(ones_v, acc_sh.at[idx_v.at[pl.ds(j * K, K)]], add=True)
        return 0

    lax.fori_loop(0, ET // K, body, 0)
    pltpu.sync_copy(ones_v.at[pl.ds(0, ET - (ET // K) * K)],
                    acc_sh.at[idx_v.at[pl.ds((ET // K) * K, ET - (ET // K) * K)]],
                    add=True)
    plsc.subcore_barrier()

    pltpu.sync_copy(acc_sh.at[pl.ds(s * RPT, RPT)],
                    out_hbm.at[c, pl.ds(s * RPT, RPT)])


@functools.partial(
    pl.kernel,
    mesh=_sc_mesh(),
    out_type=jax.ShapeDtypeStruct((NC, N, H), BF),
    scratch_types=[
        pltpu.VMEM_SHARED((N, H), BF),
        pltpu.VMEM((ET,), jnp.int32),
        pltpu.VMEM((ET,), jnp.int32),
        pltpu.VMEM((K, H), BF),
        pltpu.VMEM((K, H), BF),
        pltpu.VMEM((K, H), BF),
        pltpu.VMEM((K, H), BF),
        pltpu.SemaphoreType.DMA,
        pltpu.SemaphoreType.DMA,
        pltpu.SemaphoreType.DMA,
        pltpu.SemaphoreType.DMA,
    ],
    compiler_params=pltpu.CompilerParams(use_tc_tiling_on_sc=False),
)
def _agg_kernel(edges_hbm, y_hbm, out_hbm,
                acc_sh, src_v, dst_v, rows0, rows1, rows2, rows3,
                semA, semB, semC, semD):
    c = lax.axis_index("c")
    s = lax.axis_index("s")
    tile = c * NS + s

    def fill_row(i, _):
        def fill_lane(k, _):
            rows0[i, pl.ds(k * 2 * L, 2 * L)] = jnp.zeros((2 * L,), BF)
            return 0

        lax.fori_loop(0, H // (2 * L), fill_lane, 0)
        return 0

    lax.fori_loop(0, K, fill_row, 0)

    def zcopy(m, _):
        pltpu.sync_copy(rows0, acc_sh.at[pl.ds(s * RPT + m * K, K)])
        return 0

    lax.fori_loop(0, RPT // K, zcopy, 0)
    pltpu.sync_copy(rows0.at[pl.ds(0, RPT - (RPT // K) * K)],
                    acc_sh.at[pl.ds(s * RPT + (RPT // K) * K,
                                    RPT - (RPT // K) * K)])
    plsc.subcore_barrier()

    pltpu.sync_copy(edges_hbm.at[0, pl.ds(tile * ET, ET)], src_v)
    pltpu.sync_copy(edges_hbm.at[1, pl.ds(tile * ET, ET)], dst_v)

    bufs = (rows0, rows1, rows2, rows3)
    sems = (semA, semB, semC, semD)
    for b in range(3):
        pltpu.async_copy(y_hbm.at[src_v.at[pl.ds(b * K, K)]], bufs[b],
                         sems[b])

    def body(i, _):
        for b in range(4):
            a = 4 * i + b
            pltpu.make_async_copy(
                y_hbm.at[src_v.at[pl.ds(a * K, K)]], bufs[b], sems[b]).wait()

            @pl.when(a + 3 < NF)
            def _():
                pltpu.async_copy(
                    y_hbm.at[src_v.at[pl.ds((a + 3) * K, K)]],
                    bufs[(b + 3) % 4], sems[(b + 3) % 4])

            pltpu.sync_copy(bufs[b], acc_sh.at[dst_v.at[pl.ds(a * K, K)]],
                            add=True)
        return 0

    lax.fori_loop(0, NF // 4, body, 0)
    for a in range(4 * (NF // 4), NF):
        b = a % 4
        pltpu.make_async_copy(
            y_hbm.at[src_v.at[pl.ds(a * K, K)]], bufs[b], sems[b]).wait()
        pltpu.sync_copy(bufs[b], acc_sh.at[dst_v.at[pl.ds(a * K, K)]],
                        add=True)
    pltpu.async_copy(y_hbm.at[src_v.at[pl.ds(NF * K, TAIL)]],
                     rows0.at[pl.ds(0, TAIL)], semA)
    pltpu.make_async_copy(y_hbm.at[src_v.at[pl.ds(NF * K, TAIL)]],
                          rows0.at[pl.ds(0, TAIL)], semA).wait()
    pltpu.sync_copy(rows0.at[pl.ds(0, TAIL)],
                    acc_sh.at[dst_v.at[pl.ds(NF * K, TAIL)]], add=True)

    plsc.subcore_barrier()
    pltpu.sync_copy(acc_sh.at[pl.ds(s * RPT, RPT)],
                    out_hbm.at[c, pl.ds(s * RPT, RPT)])


def _mid_body(x_ref, w1_ref, degp_ref, y_ref):
    xw = jnp.dot(x_ref[...], w1_ref[...], preferred_element_type=jnp.float32)
    deg = degp_ref[0, :, 0:1] + degp_ref[1, :, 0:1] + 1.0
    dis = lax.rsqrt(deg)
    y_ref[...] = xw * dis


def _head_body(degp_ref, acc_ref, y_ref, b1_ref, w2_ref, b2_ref, out_ref):
    deg = degp_ref[0, :, 0:1] + degp_ref[1, :, 0:1] + 1.0
    dis = lax.rsqrt(deg)
    acc = acc_ref[...] + y_ref[...]
    h = jnp.maximum(dis * acc + b1_ref[...], 0.0)
    out_ref[...] = jnp.sum(h * w2_ref[...], axis=1, keepdims=True) + b2_ref[...]


def kernel(x, edge_index, W1, b1, W2, b2):
    deg_parts = _deg_kernel(edge_index)

    y = pl.pallas_call(
        _mid_body,
        grid=(N // BR,),
        in_specs=[
            pl.BlockSpec((BR, D), lambda i: (i, 0)),
            pl.BlockSpec((D, H), lambda i: (0, 0)),
            pl.BlockSpec((NC, BR, L), lambda i: (0, i, 0)),
        ],
        out_specs=pl.BlockSpec((BR, H), lambda i: (i, 0)),
        out_shape=jax.ShapeDtypeStruct((N, H), jnp.float32),
    )(x, W1, deg_parts)

    acc_parts = _agg_kernel(edge_index, y.astype(BF))
    acc = (acc_parts[0].astype(jnp.float32)
           + acc_parts[1].astype(jnp.float32))

    out2d = pl.pallas_call(
        _head_body,
        grid=(N // BR,),
        in_specs=[
            pl.BlockSpec((NC, BR, L), lambda i: (0, i, 0)),
            pl.BlockSpec((BR, H), lambda i: (i, 0)),
            pl.BlockSpec((BR, H), lambda i: (i, 0)),
            pl.BlockSpec((1, H), lambda i: (0, 0)),
            pl.BlockSpec((1, H), lambda i: (0, 0)),
            pl.BlockSpec((1, 1), lambda i: (0, 0)),
        ],
        out_specs=pl.BlockSpec((BR, 1), lambda i: (i, 0)),
        out_shape=jax.ShapeDtypeStruct((N, 1), jnp.float32),
    )(deg_parts, acc, y, b1.reshape(1, H), W2.reshape(1, H),
      b2.reshape(1, 1))

    return out2d.reshape(N)
